# Initial kernel scaffold; baseline (speedup 1.0000x reference)
import functools

import jax
import jax.numpy as jnp
from jax import lax
from jax.experimental import pallas as pl
from jax.experimental.pallas import tpu as pltpu
from jax.experimental.pallas import tpu_sc as plsc

NP = 10240
D = 256
H = 128
G = 64
BN = 1024
NB = NP // BN
NSUB = 16
CHUNK = 128
CPT = 80
EP = NSUB * CPT * CHUNK
ROWS_PER_TILE = NP // NSUB
NACC = NP + 8

_MAXNORM = 1.0 - 4e-3
_EPS = 1e-15


def _rnorm(x):
    return jnp.clip(jnp.sqrt(jnp.sum(x * x, axis=-1, keepdims=True)), _EPS, None)


def _artanh(x):
    x = jnp.clip(x, -1.0 + 1e-7, 1.0 - 1e-7)
    return 0.5 * jnp.log((1.0 + x) / (1.0 - x))


def _proj(x):
    n = _rnorm(x)
    return jnp.where(n > _MAXNORM, x / n * _MAXNORM, x)


def _expmap0(u):
    un = _rnorm(u)
    return jnp.tanh(un) * u / un


def _logmap0(p):
    pn = _rnorm(p)
    return _artanh(pn) * p / pn


def _mobius_add(x, y):
    x2 = jnp.sum(x * x, -1, keepdims=True)
    y2 = jnp.sum(y * y, -1, keepdims=True)
    xy = jnp.sum(x * y, -1, keepdims=True)
    num = (1.0 + 2.0 * xy + y2) * x + (1.0 - x2) * y
    den = 1.0 + 2.0 * xy + x2 * y2
    return num / jnp.clip(den, _EPS, None)


def _mobius_matvec(w, x):
    xn = _rnorm(x)
    mx = lax.dot_general(x, w, (((1,), (1,)), ((), ())),
                         preferred_element_type=jnp.float32)
    mxn = _rnorm(mx)
    res = jnp.tanh(mxn / xn * _artanh(xn)) * mx / mxn
    cond = jnp.max(jnp.abs(mx), axis=-1, keepdims=True) == 0.0
    return jnp.where(cond, 0.0, res)


def _hyp_linear(w, b, h):
    mv = _proj(_mobius_matvec(w, h))
    hb = _proj(_expmap0(b))
    return _proj(_mobius_add(mv, hb))


def _post_agg(agg):
    h = _proj(_expmap0(agg))
    ht = jax.nn.relu(_logmap0(h))
    return _proj(_expmap0(ht))


def _tc_in_body(x_ref, w_ref, b_ref, o0_ref, o1_ref):
    h = _proj(_expmap0(x_ref[...]))
    h = _hyp_linear(w_ref[...], b_ref[...], h)
    ht = _logmap0(h)
    o0_ref[...] = ht[:, :H]
    o1_ref[...] = ht[:, H:]


def _tc_in(xp, w, b):
    return pl.pallas_call(
        _tc_in_body,
        grid=(NB,),
        in_specs=[
            pl.BlockSpec((BN, D), lambda i: (i, 0)),
            pl.BlockSpec((D, D), lambda i: (0, 0)),
            pl.BlockSpec((1, D), lambda i: (0, 0)),
        ],
        out_specs=[
            pl.BlockSpec((BN, H), lambda i: (i, 0)),
            pl.BlockSpec((BN, H), lambda i: (i, 0)),
        ],
        out_shape=[jax.ShapeDtypeStruct((NP, H), jnp.float32)] * 2,
    )(xp, w, b)


def _tc_mid_body(a0_ref, a1_ref, w_ref, b_ref, o0_ref, o1_ref):
    agg = jnp.concatenate([a0_ref[...], a1_ref[...]], axis=1)
    h = _post_agg(agg)
    h = _hyp_linear(w_ref[...], b_ref[...], h)
    ht = _logmap0(h)
    o0_ref[...] = ht[:, :H]
    o1_ref[...] = ht[:, H:]


def _tc_mid(a0, a1, w, b):
    return pl.pallas_call(
        _tc_mid_body,
        grid=(NB,),
        in_specs=[
            pl.BlockSpec((BN, H), lambda i: (i, 0)),
            pl.BlockSpec((BN, H), lambda i: (i, 0)),
            pl.BlockSpec((D, D), lambda i: (0, 0)),
            pl.BlockSpec((1, D), lambda i: (0, 0)),
        ],
        out_specs=[
            pl.BlockSpec((BN, H), lambda i: (i, 0)),
            pl.BlockSpec((BN, H), lambda i: (i, 0)),
        ],
        out_shape=[jax.ShapeDtypeStruct((NP, H), jnp.float32)] * 2,
    )(a0, a1, w, b)


def _tc_pool_body(a0_ref, a1_ref, batch_ref, gw_ref, out_ref,
                  smax_s, den_s, num_s):
    p = pl.program_id(0)
    j = pl.program_id(1)

    agg = jnp.concatenate([a0_ref[...], a1_ref[...]], axis=1)
    h = _post_agg(agg)
    gw = gw_ref[...]
    gl = lax.dot_general(gw, h, (((1,), (1,)), ((), ())),
                         preferred_element_type=jnp.float32)
    b2d = batch_ref[...].reshape(1, BN)
    seg = lax.broadcasted_iota(jnp.int32, (G, BN), 0)
    mask = seg == b2d

    @pl.when(jnp.logical_and(p == 0, j == 0))
    def _():
        smax_s[...] = jnp.full_like(smax_s[...], -1e30)

    @pl.when(p == 0)
    def _():
        bm = jnp.max(jnp.where(mask, gl, -1e30), axis=1, keepdims=True)
        smax_s[...] = jnp.maximum(smax_s[...], bm)

    @pl.when(jnp.logical_and(p == 1, j == 0))
    def _():
        den_s[...] = jnp.zeros_like(den_s[...])
        num_s[...] = jnp.zeros_like(num_s[...])

    @pl.when(p == 1)
    def _():
        m = jnp.max(smax_s[...], axis=1, keepdims=True)
        e = jnp.where(mask, jnp.exp(gl - m), 0.0)
        den_s[...] += jnp.sum(e, axis=1, keepdims=True)
        num_s[...] += lax.dot_general(e, h, (((1,), (0,)), ((), ())),
                                      preferred_element_type=jnp.float32)

    @pl.when(jnp.logical_and(p == 1, j == NB - 1))
    def _():
        den = jnp.max(den_s[...], axis=1, keepdims=True)
        out_ref[...] = num_s[...] / (den + 1e-16)


def _tc_pool(a0, a1, batch3, gw):
    return pl.pallas_call(
        _tc_pool_body,
        grid=(2, NB),
        in_specs=[
            pl.BlockSpec((BN, H), lambda p, j: (j, 0)),
            pl.BlockSpec((BN, H), lambda p, j: (j, 0)),
            pl.BlockSpec((1, 1, BN), lambda p, j: (j, 0, 0)),
            pl.BlockSpec((1, D), lambda p, j: (0, 0)),
        ],
        out_specs=pl.BlockSpec((G, D), lambda p, j: (0, 0)),
        out_shape=jax.ShapeDtypeStruct((G, D), jnp.float32),
        scratch_shapes=[
            pltpu.VMEM((G, 128), jnp.float32),
            pltpu.VMEM((G, 128), jnp.float32),
            pltpu.VMEM((G, D), jnp.float32),
        ],
    )(a0, a1, batch3, gw)


def _sc_agg_body(ht0, ht1, src_hbm, dst_hbm, zeros_hbm, o0, o1,
                 src_v, dst_v, rows_v, acc, sem):
    c = lax.axis_index("c")
    s = lax.axis_index("s")

    pltpu.sync_copy(zeros_hbm, acc.at[pl.ds(s * ROWS_PER_TILE, ROWS_PER_TILE)])

    def run(table, out_ref):
        pltpu.sync_copy(src_hbm.at[pl.ds(s * CPT, CPT)], src_v)
        pltpu.sync_copy(dst_hbm.at[pl.ds(s * CPT, CPT)], dst_v)
        plsc.subcore_barrier()

        @functools.partial(pl.loop, 0, CPT)
        def _(j):
            pltpu.async_copy(table.at[src_v.at[j]], rows_v, sem).wait()
            pltpu.sync_copy(rows_v, acc.at[dst_v.at[j]], add=True)

        plsc.subcore_barrier()
        base = s * ROWS_PER_TILE
        pltpu.sync_copy(acc.at[pl.ds(base, ROWS_PER_TILE)],
                        out_ref.at[pl.ds(base, ROWS_PER_TILE)])

    @pl.when(c == 0)
    def _():
        run(ht0, o0)

    @pl.when(c == 1)
    def _():
        run(ht1, o1)


@functools.partial(
    pl.kernel,
    out_type=[jax.ShapeDtypeStruct((NP, H), jnp.float32)] * 2,
    mesh=plsc.VectorSubcoreMesh(core_axis_name="c", subcore_axis_name="s"),
    scratch_types=[
        pltpu.VMEM((CPT, CHUNK), jnp.int32),
        pltpu.VMEM((CPT, CHUNK), jnp.int32),
        pltpu.VMEM((CHUNK, H), jnp.float32),
        pltpu.VMEM_SHARED((NACC, H), jnp.float32),
        pltpu.SemaphoreType.DMA,
    ],
)
def _sc_agg(ht0, ht1, src2d, dst2d, zeros, o0, o1, src_v, dst_v, rows_v, acc, sem):
    _sc_agg_body(ht0, ht1, src2d, dst2d, zeros, o0, o1,
                 src_v, dst_v, rows_v, acc, sem)


def kernel(x, edge_index, batch, W1, b1, W2, b2, gate_w, gate_b):
    n = x.shape[0]
    e = edge_index.shape[1]

    xp = jnp.zeros((NP, D), jnp.float32).at[:n].set(x)
    batchp = jnp.full((NP,), G, jnp.int32).at[:n].set(batch)
    batch3 = batchp.reshape(NB, 1, BN)

    pad = EP - e
    filler = jnp.arange(pad, dtype=jnp.int32)
    src = jnp.concatenate([edge_index[0], filler % n]).reshape(NSUB * CPT, CHUNK)
    dst = jnp.concatenate([edge_index[1], NP + (filler % 8)]).reshape(NSUB * CPT, CHUNK)
    zeros = jnp.zeros((ROWS_PER_TILE, H), jnp.float32)

    b1r = b1.reshape(1, D)
    b2r = b2.reshape(1, D)
    gw = gate_w.reshape(1, D)
    del gate_b

    ht0, ht1 = _tc_in(xp, W1, b1r)
    a0, a1 = _sc_agg(ht0, ht1, src, dst, zeros)
    ht0, ht1 = _tc_mid(a0, a1, W2, b2r)
    a0, a1 = _sc_agg(ht0, ht1, src, dst, zeros)
    return _tc_pool(a0, a1, batch3, gw)


# speedup vs baseline: 5.3095x; 5.3095x over previous
import functools

import jax
import jax.numpy as jnp
from jax import lax
from jax.experimental import pallas as pl
from jax.experimental.pallas import tpu as pltpu
from jax.experimental.pallas import tpu_sc as plsc

NP = 10240
D = 256
H = 128
G = 64
BN = 1024
NB = NP // BN
NSUB = 16
CHUNK = 128
CPT = 80
EP = NSUB * CPT * CHUNK
ROWS_PER_TILE = NP // NSUB
NACC = NP + 8

_MAXNORM = 1.0 - 4e-3
_EPS = 1e-15


def _rnorm(x):
    return jnp.clip(jnp.sqrt(jnp.sum(x * x, axis=-1, keepdims=True)), _EPS, None)


def _artanh(x):
    x = jnp.clip(x, -1.0 + 1e-7, 1.0 - 1e-7)
    return 0.5 * jnp.log((1.0 + x) / (1.0 - x))


def _proj(x):
    n = _rnorm(x)
    return jnp.where(n > _MAXNORM, x / n * _MAXNORM, x)


def _expmap0(u):
    un = _rnorm(u)
    return jnp.tanh(un) * u / un


def _logmap0(p):
    pn = _rnorm(p)
    return _artanh(pn) * p / pn


def _mobius_add(x, y):
    x2 = jnp.sum(x * x, -1, keepdims=True)
    y2 = jnp.sum(y * y, -1, keepdims=True)
    xy = jnp.sum(x * y, -1, keepdims=True)
    num = (1.0 + 2.0 * xy + y2) * x + (1.0 - x2) * y
    den = 1.0 + 2.0 * xy + x2 * y2
    return num / jnp.clip(den, _EPS, None)


def _mobius_matvec(w, x):
    xn = _rnorm(x)
    mx = lax.dot_general(x, w, (((1,), (1,)), ((), ())),
                         preferred_element_type=jnp.float32)
    mxn = _rnorm(mx)
    res = jnp.tanh(mxn / xn * _artanh(xn)) * mx / mxn
    cond = jnp.max(jnp.abs(mx), axis=-1, keepdims=True) == 0.0
    return jnp.where(cond, 0.0, res)


def _hyp_linear(w, b, h):
    mv = _proj(_mobius_matvec(w, h))
    hb = _proj(_expmap0(b))
    return _proj(_mobius_add(mv, hb))


def _post_agg(agg):
    h = _proj(_expmap0(agg))
    ht = jax.nn.relu(_logmap0(h))
    return _proj(_expmap0(ht))


def _tc_in_body(x_ref, w_ref, b_ref, o0_ref, o1_ref):
    h = _proj(_expmap0(x_ref[...]))
    h = _hyp_linear(w_ref[...], b_ref[...], h)
    ht = _logmap0(h)
    o0_ref[...] = ht[:, :H]
    o1_ref[...] = ht[:, H:]


def _tc_in(xp, w, b):
    return pl.pallas_call(
        _tc_in_body,
        grid=(NB,),
        in_specs=[
            pl.BlockSpec((BN, D), lambda i: (i, 0)),
            pl.BlockSpec((D, D), lambda i: (0, 0)),
            pl.BlockSpec((1, D), lambda i: (0, 0)),
        ],
        out_specs=[
            pl.BlockSpec((BN, H), lambda i: (i, 0)),
            pl.BlockSpec((BN, H), lambda i: (i, 0)),
        ],
        out_shape=[jax.ShapeDtypeStruct((NP, H), jnp.float32)] * 2,
    )(xp, w, b)


def _tc_mid_body(a0_ref, a1_ref, w_ref, b_ref, o0_ref, o1_ref):
    agg = jnp.concatenate([a0_ref[...], a1_ref[...]], axis=1)
    h = _post_agg(agg)
    h = _hyp_linear(w_ref[...], b_ref[...], h)
    ht = _logmap0(h)
    o0_ref[...] = ht[:, :H]
    o1_ref[...] = ht[:, H:]


def _tc_mid(a0, a1, w, b):
    return pl.pallas_call(
        _tc_mid_body,
        grid=(NB,),
        in_specs=[
            pl.BlockSpec((BN, H), lambda i: (i, 0)),
            pl.BlockSpec((BN, H), lambda i: (i, 0)),
            pl.BlockSpec((D, D), lambda i: (0, 0)),
            pl.BlockSpec((1, D), lambda i: (0, 0)),
        ],
        out_specs=[
            pl.BlockSpec((BN, H), lambda i: (i, 0)),
            pl.BlockSpec((BN, H), lambda i: (i, 0)),
        ],
        out_shape=[jax.ShapeDtypeStruct((NP, H), jnp.float32)] * 2,
    )(a0, a1, w, b)


def _tc_pool_body(a0_ref, a1_ref, batch_ref, gw_ref, out_ref,
                  smax_s, den_s, num_s):
    p = pl.program_id(0)
    j = pl.program_id(1)

    agg = jnp.concatenate([a0_ref[...], a1_ref[...]], axis=1)
    h = _post_agg(agg)
    gw = gw_ref[...]
    gl = lax.dot_general(gw, h, (((1,), (1,)), ((), ())),
                         preferred_element_type=jnp.float32)
    b2d = batch_ref[...].reshape(1, BN)
    seg = lax.broadcasted_iota(jnp.int32, (G, BN), 0)
    mask = seg == b2d

    @pl.when(jnp.logical_and(p == 0, j == 0))
    def _():
        smax_s[...] = jnp.full_like(smax_s[...], -1e30)

    @pl.when(p == 0)
    def _():
        bm = jnp.max(jnp.where(mask, gl, -1e30), axis=1, keepdims=True)
        smax_s[...] = jnp.maximum(smax_s[...], bm)

    @pl.when(jnp.logical_and(p == 1, j == 0))
    def _():
        den_s[...] = jnp.zeros_like(den_s[...])
        num_s[...] = jnp.zeros_like(num_s[...])

    @pl.when(p == 1)
    def _():
        m = jnp.max(smax_s[...], axis=1, keepdims=True)
        e = jnp.where(mask, jnp.exp(gl - m), 0.0)
        den_s[...] += jnp.sum(e, axis=1, keepdims=True)
        num_s[...] += lax.dot_general(e, h, (((1,), (0,)), ((), ())),
                                      preferred_element_type=jnp.float32)

    @pl.when(jnp.logical_and(p == 1, j == NB - 1))
    def _():
        den = jnp.max(den_s[...], axis=1, keepdims=True)
        out_ref[...] = num_s[...] / (den + 1e-16)


def _tc_pool(a0, a1, batch3, gw):
    return pl.pallas_call(
        _tc_pool_body,
        grid=(2, NB),
        in_specs=[
            pl.BlockSpec((BN, H), lambda p, j: (j, 0)),
            pl.BlockSpec((BN, H), lambda p, j: (j, 0)),
            pl.BlockSpec((1, 1, BN), lambda p, j: (j, 0, 0)),
            pl.BlockSpec((1, D), lambda p, j: (0, 0)),
        ],
        out_specs=pl.BlockSpec((G, D), lambda p, j: (0, 0)),
        out_shape=jax.ShapeDtypeStruct((G, D), jnp.float32),
        scratch_shapes=[
            pltpu.VMEM((G, 128), jnp.float32),
            pltpu.VMEM((G, 128), jnp.float32),
            pltpu.VMEM((G, D), jnp.float32),
        ],
    )(a0, a1, batch3, gw)


def _sc_agg_body(ht0, ht1, src_hbm, dst_hbm, zeros_hbm, o0, o1,
                 src_v, dst_v, rows_v, acc, sem):
    c = lax.axis_index("c")
    s = lax.axis_index("s")

    pltpu.sync_copy(zeros_hbm, acc.at[pl.ds(s * ROWS_PER_TILE, ROWS_PER_TILE)])

    def run(table, out_ref):
        pltpu.sync_copy(src_hbm.at[pl.ds(s * CPT, CPT)], src_v)
        pltpu.sync_copy(dst_hbm.at[pl.ds(s * CPT, CPT)], dst_v)
        plsc.subcore_barrier()

        @pl.loop(0, CPT)
        def _(j):
            pltpu.async_copy(table.at[src_v.at[j]], rows_v, sem).wait()
            pltpu.sync_copy(rows_v, acc.at[dst_v.at[j]], add=True)

        plsc.subcore_barrier()
        base = s * ROWS_PER_TILE
        pltpu.sync_copy(acc.at[pl.ds(base, ROWS_PER_TILE)],
                        out_ref.at[pl.ds(base, ROWS_PER_TILE)])

    @pl.when(c == 0)
    def _():
        run(ht0, o0)

    @pl.when(c == 1)
    def _():
        run(ht1, o1)


@functools.cache
def _make_sc_agg():
    return pl.kernel(
        _sc_agg_body,
        out_type=[jax.ShapeDtypeStruct((NP, H), jnp.float32)] * 2,
        mesh=plsc.VectorSubcoreMesh(core_axis_name="c", subcore_axis_name="s"),
        scratch_types=[
            pltpu.VMEM((CPT, CHUNK), jnp.int32),
            pltpu.VMEM((CPT, CHUNK), jnp.int32),
            pltpu.VMEM((CHUNK, H), jnp.float32),
            pltpu.VMEM_SHARED((NACC, H), jnp.float32),
            pltpu.SemaphoreType.DMA,
        ],
    )


def _sc_agg(ht0, ht1, src2d, dst2d, zeros):
    return _make_sc_agg()(ht0, ht1, src2d, dst2d, zeros)


def kernel(x, edge_index, batch, W1, b1, W2, b2, gate_w, gate_b):
    n = x.shape[0]
    e = edge_index.shape[1]

    xp = jnp.zeros((NP, D), jnp.float32).at[:n].set(x)
    batchp = jnp.full((NP,), G, jnp.int32).at[:n].set(batch)
    batch3 = batchp.reshape(NB, 1, BN)

    pad = EP - e
    filler = jnp.arange(pad, dtype=jnp.int32)
    src = jnp.concatenate([edge_index[0], filler % n]).reshape(NSUB * CPT, CHUNK)
    dst = jnp.concatenate([edge_index[1], NP + (filler % 8)]).reshape(NSUB * CPT, CHUNK)
    zeros = jnp.zeros((ROWS_PER_TILE, H), jnp.float32)

    b1r = b1.reshape(1, D)
    b2r = b2.reshape(1, D)
    gw = gate_w.reshape(1, D)
    del gate_b

    ht0, ht1 = _tc_in(xp, W1, b1r)
    a0, a1 = _sc_agg(ht0, ht1, src, dst, zeros)
    ht0, ht1 = _tc_mid(a0, a1, W2, b2r)
    a0, a1 = _sc_agg(ht0, ht1, src, dst, zeros)
    return _tc_pool(a0, a1, batch3, gw)


# speedup vs baseline: 6.8806x; 1.2959x over previous
import functools

import jax
import jax.numpy as jnp
from jax import lax
from jax.experimental import pallas as pl
from jax.experimental.pallas import tpu as pltpu
from jax.experimental.pallas import tpu_sc as plsc

NP = 10240
D = 256
H = 128
G = 64
BN = 1024
NB = NP // BN
NSUB = 16
CHUNK = 128
CPT = 80
IB = 16
NBLK = CPT // IB
EP = NSUB * CPT * CHUNK
ROWS_PER_TILE = NP // NSUB
NACC = NP + 8

_MAXNORM = 1.0 - 4e-3
_EPS = 1e-15


def _rnorm(x):
    return jnp.clip(jnp.sqrt(jnp.sum(x * x, axis=-1, keepdims=True)), _EPS, None)


def _artanh(x):
    x = jnp.clip(x, -1.0 + 1e-7, 1.0 - 1e-7)
    return 0.5 * jnp.log((1.0 + x) / (1.0 - x))


def _proj(x):
    n = _rnorm(x)
    return jnp.where(n > _MAXNORM, x / n * _MAXNORM, x)


def _expmap0(u):
    un = _rnorm(u)
    return jnp.tanh(un) * u / un


def _logmap0(p):
    pn = _rnorm(p)
    return _artanh(pn) * p / pn


def _mobius_add(x, y):
    x2 = jnp.sum(x * x, -1, keepdims=True)
    y2 = jnp.sum(y * y, -1, keepdims=True)
    xy = jnp.sum(x * y, -1, keepdims=True)
    num = (1.0 + 2.0 * xy + y2) * x + (1.0 - x2) * y
    den = 1.0 + 2.0 * xy + x2 * y2
    return num / jnp.clip(den, _EPS, None)


def _mobius_matvec(w, x):
    xn = _rnorm(x)
    mx = lax.dot_general(x, w, (((1,), (1,)), ((), ())),
                         preferred_element_type=jnp.float32)
    mxn = _rnorm(mx)
    res = jnp.tanh(mxn / xn * _artanh(xn)) * mx / mxn
    cond = jnp.max(jnp.abs(mx), axis=-1, keepdims=True) == 0.0
    return jnp.where(cond, 0.0, res)


def _hyp_linear(w, b, h):
    mv = _proj(_mobius_matvec(w, h))
    hb = _proj(_expmap0(b))
    return _proj(_mobius_add(mv, hb))


def _post_agg(agg):
    h = _proj(_expmap0(agg))
    ht = jax.nn.relu(_logmap0(h))
    return _proj(_expmap0(ht))


def _tc_in_body(x_ref, w_ref, b_ref, o0_ref, o1_ref):
    h = _proj(_expmap0(x_ref[...]))
    h = _hyp_linear(w_ref[...], b_ref[...], h)
    ht = _logmap0(h)
    o0_ref[...] = ht[:, :H]
    o1_ref[...] = ht[:, H:]


def _tc_in(xp, w, b):
    return pl.pallas_call(
        _tc_in_body,
        grid=(NB,),
        in_specs=[
            pl.BlockSpec((BN, D), lambda i: (i, 0)),
            pl.BlockSpec((D, D), lambda i: (0, 0)),
            pl.BlockSpec((1, D), lambda i: (0, 0)),
        ],
        out_specs=[
            pl.BlockSpec((BN, H), lambda i: (i, 0)),
            pl.BlockSpec((BN, H), lambda i: (i, 0)),
        ],
        out_shape=[jax.ShapeDtypeStruct((NP, H), jnp.float32)] * 2,
    )(xp, w, b)


def _tc_mid_body(a0_ref, a1_ref, w_ref, b_ref, o0_ref, o1_ref):
    agg = jnp.concatenate([a0_ref[...], a1_ref[...]], axis=1)
    h = _post_agg(agg)
    h = _hyp_linear(w_ref[...], b_ref[...], h)
    ht = _logmap0(h)
    o0_ref[...] = ht[:, :H]
    o1_ref[...] = ht[:, H:]


def _tc_mid(a0, a1, w, b):
    return pl.pallas_call(
        _tc_mid_body,
        grid=(NB,),
        in_specs=[
            pl.BlockSpec((BN, H), lambda i: (i, 0)),
            pl.BlockSpec((BN, H), lambda i: (i, 0)),
            pl.BlockSpec((D, D), lambda i: (0, 0)),
            pl.BlockSpec((1, D), lambda i: (0, 0)),
        ],
        out_specs=[
            pl.BlockSpec((BN, H), lambda i: (i, 0)),
            pl.BlockSpec((BN, H), lambda i: (i, 0)),
        ],
        out_shape=[jax.ShapeDtypeStruct((NP, H), jnp.float32)] * 2,
    )(a0, a1, w, b)


def _tc_pool_body(a0_ref, a1_ref, batch_ref, gw_ref, out_ref,
                  smax_s, den_s, num_s):
    p = pl.program_id(0)
    j = pl.program_id(1)

    agg = jnp.concatenate([a0_ref[...], a1_ref[...]], axis=1)
    h = _post_agg(agg)
    gw = gw_ref[...]
    gl = lax.dot_general(gw, h, (((1,), (1,)), ((), ())),
                         preferred_element_type=jnp.float32)
    b2d = batch_ref[...].reshape(1, BN)
    seg = lax.broadcasted_iota(jnp.int32, (G, BN), 0)
    mask = seg == b2d

    @pl.when(jnp.logical_and(p == 0, j == 0))
    def _():
        smax_s[...] = jnp.full_like(smax_s[...], -1e30)

    @pl.when(p == 0)
    def _():
        bm = jnp.max(jnp.where(mask, gl, -1e30), axis=1, keepdims=True)
        smax_s[...] = jnp.maximum(smax_s[...], bm)

    @pl.when(jnp.logical_and(p == 1, j == 0))
    def _():
        den_s[...] = jnp.zeros_like(den_s[...])
        num_s[...] = jnp.zeros_like(num_s[...])

    @pl.when(p == 1)
    def _():
        m = jnp.max(smax_s[...], axis=1, keepdims=True)
        e = jnp.where(mask, jnp.exp(gl - m), 0.0)
        den_s[...] += jnp.sum(e, axis=1, keepdims=True)
        num_s[...] += lax.dot_general(e, h, (((1,), (0,)), ((), ())),
                                      preferred_element_type=jnp.float32)

    @pl.when(jnp.logical_and(p == 1, j == NB - 1))
    def _():
        den = jnp.max(den_s[...], axis=1, keepdims=True)
        out_ref[...] = num_s[...] / (den + 1e-16)


def _tc_pool(a0, a1, batch3, gw):
    return pl.pallas_call(
        _tc_pool_body,
        grid=(2, NB),
        in_specs=[
            pl.BlockSpec((BN, H), lambda p, j: (j, 0)),
            pl.BlockSpec((BN, H), lambda p, j: (j, 0)),
            pl.BlockSpec((1, 1, BN), lambda p, j: (j, 0, 0)),
            pl.BlockSpec((1, D), lambda p, j: (0, 0)),
        ],
        out_specs=pl.BlockSpec((G, D), lambda p, j: (0, 0)),
        out_shape=jax.ShapeDtypeStruct((G, D), jnp.float32),
        scratch_shapes=[
            pltpu.VMEM((G, 128), jnp.float32),
            pltpu.VMEM((G, 128), jnp.float32),
            pltpu.VMEM((G, D), jnp.float32),
        ],
    )(a0, a1, batch3, gw)


def _sc_agg_body(ht0, ht1, src_hbm, dst_hbm, zeros_hbm, o0, o1,
                 src_v, dst_v, rows0, rows1, acc, sem0, sem1):
    c = lax.axis_index("c")
    s = lax.axis_index("s")

    pltpu.sync_copy(zeros_hbm, acc.at[pl.ds(s * ROWS_PER_TILE, ROWS_PER_TILE)])

    def run(table, out_ref):
        plsc.subcore_barrier()

        @pl.loop(0, NBLK)
        def _(k):
            blk = s * CPT + k * IB
            pltpu.sync_copy(src_hbm.at[pl.ds(blk, IB)], src_v)
            pltpu.sync_copy(dst_hbm.at[pl.ds(blk, IB)], dst_v)
            pltpu.async_copy(table.at[src_v.at[0]], rows0, sem0)

            @pl.loop(0, IB // 2)
            def _(i):
                j0 = 2 * i
                pltpu.async_copy(table.at[src_v.at[j0 + 1]], rows1, sem1)
                pltpu.make_async_copy(table.at[src_v.at[j0]], rows0, sem0).wait()
                pltpu.sync_copy(rows0, acc.at[dst_v.at[j0]], add=True)

                @pl.when(j0 + 2 < IB)
                def _():
                    pltpu.async_copy(table.at[src_v.at[j0 + 2]], rows0, sem0)

                pltpu.make_async_copy(table.at[src_v.at[j0 + 1]], rows1, sem1).wait()
                pltpu.sync_copy(rows1, acc.at[dst_v.at[j0 + 1]], add=True)

        plsc.subcore_barrier()
        base = s * ROWS_PER_TILE
        pltpu.sync_copy(acc.at[pl.ds(base, ROWS_PER_TILE)],
                        out_ref.at[pl.ds(base, ROWS_PER_TILE)])

    @pl.when(c == 0)
    def _():
        run(ht0, o0)

    @pl.when(c == 1)
    def _():
        run(ht1, o1)


@functools.cache
def _make_sc_agg():
    return pl.kernel(
        _sc_agg_body,
        out_type=[jax.ShapeDtypeStruct((NP, H), jnp.float32)] * 2,
        mesh=plsc.VectorSubcoreMesh(core_axis_name="c", subcore_axis_name="s"),
        scratch_types=[
            pltpu.VMEM((IB, CHUNK), jnp.int32),
            pltpu.VMEM((IB, CHUNK), jnp.int32),
            pltpu.VMEM((CHUNK, H), jnp.float32),
            pltpu.VMEM((CHUNK, H), jnp.float32),
            pltpu.VMEM_SHARED((NACC, H), jnp.float32),
            pltpu.SemaphoreType.DMA,
            pltpu.SemaphoreType.DMA,
        ],
    )


def _sc_agg(ht0, ht1, src2d, dst2d, zeros):
    return _make_sc_agg()(ht0, ht1, src2d, dst2d, zeros)


def kernel(x, edge_index, batch, W1, b1, W2, b2, gate_w, gate_b):
    n = x.shape[0]
    e = edge_index.shape[1]

    xp = jnp.zeros((NP, D), jnp.float32).at[:n].set(x)
    batchp = jnp.full((NP,), G, jnp.int32).at[:n].set(batch)
    batch3 = batchp.reshape(NB, 1, BN)

    pad = EP - e
    filler = jnp.arange(pad, dtype=jnp.int32)
    src = jnp.concatenate([edge_index[0], filler % n]).reshape(NSUB * CPT, CHUNK)
    dst = jnp.concatenate([edge_index[1], NP + (filler % 8)]).reshape(NSUB * CPT, CHUNK)
    zeros = jnp.zeros((ROWS_PER_TILE, H), jnp.float32)

    b1r = b1.reshape(1, D)
    b2r = b2.reshape(1, D)
    gw = gate_w.reshape(1, D)
    del gate_b

    ht0, ht1 = _tc_in(xp, W1, b1r)
    a0, a1 = _sc_agg(ht0, ht1, src, dst, zeros)
    ht0, ht1 = _tc_mid(a0, a1, W2, b2r)
    a0, a1 = _sc_agg(ht0, ht1, src, dst, zeros)
    return _tc_pool(a0, a1, batch3, gw)


# speedup vs baseline: 7.5240x; 1.0935x over previous
import functools

import jax
import jax.numpy as jnp
from jax import lax
from jax.experimental import pallas as pl
from jax.experimental.pallas import tpu as pltpu
from jax.experimental.pallas import tpu_sc as plsc

NP = 10240
D = 256
H = 128
G = 64
BN = 1024
NB = NP // BN
NSUB = 16
CHUNK = 128
CPT = 80
IB = 16
NBLK = CPT // IB
EP = NSUB * CPT * CHUNK
ROWS_PER_TILE = NP // NSUB
NACC = NP + 8

_MAXNORM = 1.0 - 4e-3
_EPS = 1e-15


def _rnorm(x):
    return jnp.maximum(jnp.sqrt(jnp.sum(x * x, axis=-1, keepdims=True)), _EPS)


def _artanh(x):
    x = jnp.clip(x, -1.0 + 1e-7, 1.0 - 1e-7)
    return 0.5 * jnp.log((1.0 + x) / (1.0 - x))


def _proj(x):
    n = _rnorm(x)
    return x * jnp.minimum(1.0, _MAXNORM / n)


def _proj_expmap0(u):
    un = _rnorm(u)
    return u * (jnp.minimum(jnp.tanh(un), _MAXNORM) / un)


def _logmap0(p):
    pn = _rnorm(p)
    return p * (_artanh(pn) / pn)


def _proj_mobius_add(x, y):
    x2 = jnp.sum(x * x, -1, keepdims=True)
    y2 = jnp.sum(y * y, -1, keepdims=True)
    xy = jnp.sum(x * y, -1, keepdims=True)
    num = (1.0 + 2.0 * xy + y2) * x + (1.0 - x2) * y
    den = jnp.maximum(1.0 + 2.0 * xy + x2 * y2, _EPS)
    nn = _rnorm(num)
    return num * jnp.minimum(1.0 / den, _MAXNORM / nn)


def _proj_mobius_matvec(w, x):
    xn = _rnorm(x)
    mx = lax.dot_general(x, w, (((1,), (1,)), ((), ())),
                         preferred_element_type=jnp.float32)
    mxn = _rnorm(mx)
    return mx * (jnp.minimum(jnp.tanh(mxn / xn * _artanh(xn)), _MAXNORM) / mxn)


def _hyp_linear(w, b, h):
    mv = _proj_mobius_matvec(w, h)
    hb = _proj_expmap0(b)
    return _proj_mobius_add(mv, hb)


def _post_agg(agg):
    h = _proj_expmap0(agg)
    ht = jax.nn.relu(_logmap0(h))
    return _proj_expmap0(ht)


def _tc_in_body(x_ref, w_ref, b_ref, o0_ref, o1_ref):
    h = _proj_expmap0(x_ref[...])
    h = _hyp_linear(w_ref[...], b_ref[...], h)
    ht = _logmap0(h)
    o0_ref[...] = ht[:, :H]
    o1_ref[...] = ht[:, H:]


def _tc_in(xp, w, b):
    return pl.pallas_call(
        _tc_in_body,
        grid=(NB,),
        in_specs=[
            pl.BlockSpec((BN, D), lambda i: (i, 0)),
            pl.BlockSpec((D, D), lambda i: (0, 0)),
            pl.BlockSpec((1, D), lambda i: (0, 0)),
        ],
        out_specs=[
            pl.BlockSpec((BN, H), lambda i: (i, 0)),
            pl.BlockSpec((BN, H), lambda i: (i, 0)),
        ],
        out_shape=[jax.ShapeDtypeStruct((NP, H), jnp.float32)] * 2,
    )(xp, w, b)


def _tc_mid_body(a0_ref, a1_ref, w_ref, b_ref, o0_ref, o1_ref):
    agg = jnp.concatenate([a0_ref[...], a1_ref[...]], axis=1)
    h = _post_agg(agg)
    h = _hyp_linear(w_ref[...], b_ref[...], h)
    ht = _logmap0(h)
    o0_ref[...] = ht[:, :H]
    o1_ref[...] = ht[:, H:]


def _tc_mid(a0, a1, w, b):
    return pl.pallas_call(
        _tc_mid_body,
        grid=(NB,),
        in_specs=[
            pl.BlockSpec((BN, H), lambda i: (i, 0)),
            pl.BlockSpec((BN, H), lambda i: (i, 0)),
            pl.BlockSpec((D, D), lambda i: (0, 0)),
            pl.BlockSpec((1, D), lambda i: (0, 0)),
        ],
        out_specs=[
            pl.BlockSpec((BN, H), lambda i: (i, 0)),
            pl.BlockSpec((BN, H), lambda i: (i, 0)),
        ],
        out_shape=[jax.ShapeDtypeStruct((NP, H), jnp.float32)] * 2,
    )(a0, a1, w, b)


def _tc_pool_body(a0_ref, a1_ref, batch_ref, gw_ref, out_ref,
                  smax_s, den_s, num_s):
    p = pl.program_id(0)
    j = pl.program_id(1)

    agg = jnp.concatenate([a0_ref[...], a1_ref[...]], axis=1)
    h = _post_agg(agg)
    gw = gw_ref[...]
    gl = lax.dot_general(gw, h, (((1,), (1,)), ((), ())),
                         preferred_element_type=jnp.float32)
    b2d = batch_ref[...].reshape(1, BN)
    seg = lax.broadcasted_iota(jnp.int32, (G, BN), 0)
    mask = seg == b2d

    @pl.when(jnp.logical_and(p == 0, j == 0))
    def _():
        smax_s[...] = jnp.full_like(smax_s[...], -1e30)

    @pl.when(p == 0)
    def _():
        bm = jnp.max(jnp.where(mask, gl, -1e30), axis=1, keepdims=True)
        smax_s[...] = jnp.maximum(smax_s[...], bm)

    @pl.when(jnp.logical_and(p == 1, j == 0))
    def _():
        den_s[...] = jnp.zeros_like(den_s[...])
        num_s[...] = jnp.zeros_like(num_s[...])

    @pl.when(p == 1)
    def _():
        m = jnp.max(smax_s[...], axis=1, keepdims=True)
        e = jnp.where(mask, jnp.exp(gl - m), 0.0)
        den_s[...] += jnp.sum(e, axis=1, keepdims=True)
        num_s[...] += lax.dot_general(e, h, (((1,), (0,)), ((), ())),
                                      preferred_element_type=jnp.float32)

    @pl.when(jnp.logical_and(p == 1, j == NB - 1))
    def _():
        den = jnp.max(den_s[...], axis=1, keepdims=True)
        out_ref[...] = num_s[...] / (den + 1e-16)


def _tc_pool(a0, a1, batch3, gw):
    return pl.pallas_call(
        _tc_pool_body,
        grid=(2, NB),
        in_specs=[
            pl.BlockSpec((BN, H), lambda p, j: (j, 0)),
            pl.BlockSpec((BN, H), lambda p, j: (j, 0)),
            pl.BlockSpec((1, 1, BN), lambda p, j: (j, 0, 0)),
            pl.BlockSpec((1, D), lambda p, j: (0, 0)),
        ],
        out_specs=pl.BlockSpec((G, D), lambda p, j: (0, 0)),
        out_shape=jax.ShapeDtypeStruct((G, D), jnp.float32),
        scratch_shapes=[
            pltpu.VMEM((G, 128), jnp.float32),
            pltpu.VMEM((G, 128), jnp.float32),
            pltpu.VMEM((G, D), jnp.float32),
        ],
    )(a0, a1, batch3, gw)


def _sc_agg_body(ht0, ht1, src_hbm, dst_hbm, zeros_hbm, o0, o1,
                 src_v, dst_v, rows0, rows1, acc, sem0, sem1):
    c = lax.axis_index("c")
    s = lax.axis_index("s")

    pltpu.sync_copy(zeros_hbm, acc.at[pl.ds(s * ROWS_PER_TILE, ROWS_PER_TILE)])

    def run(table, out_ref):
        plsc.subcore_barrier()

        @pl.loop(0, NBLK)
        def _(k):
            blk = s * CPT + k * IB
            pltpu.sync_copy(src_hbm.at[pl.ds(blk, IB)], src_v)
            pltpu.sync_copy(dst_hbm.at[pl.ds(blk, IB)], dst_v)
            pltpu.async_copy(table.at[src_v.at[0]], rows0, sem0)

            @pl.loop(0, IB // 2)
            def _(i):
                j0 = 2 * i
                pltpu.async_copy(table.at[src_v.at[j0 + 1]], rows1, sem1)
                pltpu.make_async_copy(table.at[src_v.at[j0]], rows0, sem0).wait()
                pltpu.sync_copy(rows0, acc.at[dst_v.at[j0]], add=True)

                @pl.when(j0 + 2 < IB)
                def _():
                    pltpu.async_copy(table.at[src_v.at[j0 + 2]], rows0, sem0)

                pltpu.make_async_copy(table.at[src_v.at[j0 + 1]], rows1, sem1).wait()
                pltpu.sync_copy(rows1, acc.at[dst_v.at[j0 + 1]], add=True)

        plsc.subcore_barrier()
        base = s * ROWS_PER_TILE
        pltpu.sync_copy(acc.at[pl.ds(base, ROWS_PER_TILE)],
                        out_ref.at[pl.ds(base, ROWS_PER_TILE)])

    @pl.when(c == 0)
    def _():
        run(ht0, o0)

    @pl.when(c == 1)
    def _():
        run(ht1, o1)


@functools.cache
def _make_sc_agg():
    return pl.kernel(
        _sc_agg_body,
        out_type=[jax.ShapeDtypeStruct((NP, H), jnp.float32)] * 2,
        mesh=plsc.VectorSubcoreMesh(core_axis_name="c", subcore_axis_name="s"),
        scratch_types=[
            pltpu.VMEM((IB, CHUNK), jnp.int32),
            pltpu.VMEM((IB, CHUNK), jnp.int32),
            pltpu.VMEM((CHUNK, H), jnp.float32),
            pltpu.VMEM((CHUNK, H), jnp.float32),
            pltpu.VMEM_SHARED((NACC, H), jnp.float32),
            pltpu.SemaphoreType.DMA,
            pltpu.SemaphoreType.DMA,
        ],
    )


def _sc_agg(ht0, ht1, src2d, dst2d, zeros):
    return _make_sc_agg()(ht0, ht1, src2d, dst2d, zeros)


def kernel(x, edge_index, batch, W1, b1, W2, b2, gate_w, gate_b):
    n = x.shape[0]
    e = edge_index.shape[1]

    xp = jnp.zeros((NP, D), jnp.float32).at[:n].set(x)
    batchp = jnp.full((NP,), G, jnp.int32).at[:n].set(batch)
    batch3 = batchp.reshape(NB, 1, BN)

    pad = EP - e
    filler = jnp.arange(pad, dtype=jnp.int32)
    src = jnp.concatenate([edge_index[0], filler % n]).reshape(NSUB * CPT, CHUNK)
    dst = jnp.concatenate([edge_index[1], NP + (filler % 8)]).reshape(NSUB * CPT, CHUNK)
    zeros = jnp.zeros((ROWS_PER_TILE, H), jnp.float32)

    b1r = b1.reshape(1, D)
    b2r = b2.reshape(1, D)
    gw = gate_w.reshape(1, D)
    del gate_b

    ht0, ht1 = _tc_in(xp, W1, b1r)
    a0, a1 = _sc_agg(ht0, ht1, src, dst, zeros)
    ht0, ht1 = _tc_mid(a0, a1, W2, b2r)
    a0, a1 = _sc_agg(ht0, ht1, src, dst, zeros)
    return _tc_pool(a0, a1, batch3, gw)
